# baseline (speedup 1.0000x reference)
import functools

import jax
import jax.numpy as jnp
from jax import lax
from jax.experimental import pallas as pl
from jax.experimental.pallas import tpu as pltpu
from jax.experimental.pallas import tpu_sc as plsc

B = 8
N = 32768
D = 64
K = 8192

_NBLK = 32
_BN = N // _NBLK

_LANES = 16
_CHUNK = N // _LANES
_RADIX = 256
_NW = 32
_KW = K // (_NW // B)
_GC = 128


def _score_body(x_ref, w_ref, b_ref, y_ref, k_ref):
    xb = x_ref[...].reshape(B * _BN, D)
    w = w_ref[...]
    norm = jnp.sqrt(jnp.sum(w * w))
    acc = lax.dot_general(w, xb, (((1,), (1,)), ((), ())),
                          preferred_element_type=jnp.float32)
    y = (acc + b_ref[0, 0]) / norm
    y_ref[...] = y.reshape(B, _BN)
    bits = lax.bitcast_convert_type(y, jnp.int32)
    k_ref[...] = jnp.where(bits < 0, bits, ~bits & 0x7FFFFFFF).reshape(B, _BN)


def _scores(x, w, b2):
    return pl.pallas_call(
        _score_body,
        grid=(_NBLK,),
        in_specs=[
            pl.BlockSpec((B, _BN, D), lambda j: (0, j, 0)),
            pl.BlockSpec((1, D), lambda j: (0, 0)),
            pl.BlockSpec((1, 1), lambda j: (0, 0)),
        ],
        out_specs=[
            pl.BlockSpec((B, _BN), lambda j: (0, j)),
            pl.BlockSpec((B, _BN), lambda j: (0, j)),
        ],
        out_shape=[
            jax.ShapeDtypeStruct((B, N), jnp.float32),
            jax.ShapeDtypeStruct((B, N), jnp.int32),
        ],
    )(x, w, b2)


_G = 4
_NCHUNK = _LANES * _G
_CLEN = N // _NCHUNK


def _digit_pass(keys_v, hists, lane, shift, src, dst):
    ones = jnp.ones((_LANES,), jnp.int32)
    chunk_id = [lane + g * _LANES for g in range(_G)]

    def zero_body(j, c):
        for g in range(_G):
            hists[g][pl.ds(j * _LANES, _LANES)] = (
                jnp.zeros((_LANES,), jnp.int32))
        return c

    lax.fori_loop(0, _RADIX, zero_body, 0)

    def elems(i):
        if src is None:
            es = [chunk_id[g] * _CLEN + i for g in range(_G)]
        else:
            es = [plsc.load_gather(src, [chunk_id[g] * _CLEN + i])
                  for g in range(_G)]
        ks = [plsc.load_gather(keys_v, [e]) for e in es]
        addrs = []
        for k in ks:
            ku = lax.bitcast_convert_type(k, jnp.uint32)
            d = ((ku >> jnp.uint32(shift)) & jnp.uint32(0xFF))
            addrs.append(d.astype(jnp.int32) * _LANES + lane)
        return es, addrs

    def hist_body(ii, c):
        _, a0 = elems(ii * 2)
        _, a1 = elems(ii * 2 + 1)
        for g in range(_G):
            eq = (a0[g] == a1[g])
            plsc.addupdate_scatter(hists[g], [a0[g]],
                                   ones + eq.astype(jnp.int32))
            plsc.addupdate_scatter(hists[g], [a1[g]], ones,
                                   mask=jnp.logical_not(eq))
        return c

    lax.fori_loop(0, _CLEN // 2, hist_body, 0)

    def scan_body(j, carry):
        sl = pl.ds(j * _LANES, _LANES)
        hs = [hists[g][sl] for g in range(_G)]
        incs = [plsc.cumsum(h) for h in hs]
        for g in range(_G):
            hists[g][sl] = incs[g] - hs[g] + carry
            carry = carry + incs[g][_LANES - 1]
        return carry

    lax.fori_loop(0, _RADIX, scan_body, jnp.int32(0))

    def perm_body(ii, c):
        e0, a0 = elems(ii * 2)
        e1, a1 = elems(ii * 2 + 1)
        p0 = [plsc.load_gather(hists[g], [a0[g]]) for g in range(_G)]
        p1 = [plsc.load_gather(hists[g], [a1[g]]) for g in range(_G)]
        eqs = [(a0[g] == a1[g]) for g in range(_G)]
        for g in range(_G):
            plsc.store_scatter(dst, [p0[g]], e0[g])
            plsc.store_scatter(dst, [p1[g] + eqs[g].astype(jnp.int32)],
                               e1[g])
        for g in range(_G):
            plsc.addupdate_scatter(hists[g], [a0[g]],
                                   ones + eqs[g].astype(jnp.int32))
            plsc.addupdate_scatter(hists[g], [a1[g]], ones,
                                   mask=jnp.logical_not(eqs[g]))
        return c

    lax.fori_loop(0, _CLEN // 2, perm_body, 0)


def _sort_body(keys_hbm, out_hbm, keys_v, idx_a, idx_b, h0, h1, h2, h3):
    wid = lax.axis_index("s") * 2 + lax.axis_index("c")
    hists = [h0, h1, h2, h3]

    @pl.when(wid < B)
    def _():
        pltpu.sync_copy(keys_hbm.at[wid], keys_v)
        lane = lax.iota(jnp.int32, _LANES)
        _digit_pass(keys_v, hists, lane, 0, None, idx_a)
        _digit_pass(keys_v, hists, lane, 8, idx_a, idx_b)
        _digit_pass(keys_v, hists, lane, 16, idx_b, idx_a)
        _digit_pass(keys_v, hists, lane, 24, idx_a, idx_b)
        pltpu.sync_copy(idx_b.at[pl.ds(0, K)], out_hbm.at[wid])


@functools.cache
def _get_sort():
    return pl.kernel(
        _sort_body,
        out_type=jax.ShapeDtypeStruct((B, K), jnp.int32),
        compiler_params=pltpu.CompilerParams(needs_layout_passes=False,
                                             use_tc_tiling_on_sc=False),
        mesh=plsc.VectorSubcoreMesh(core_axis_name="c", subcore_axis_name="s",
                                    num_cores=2, num_subcores=16),
        scratch_types=[
            pltpu.VMEM((N,), jnp.int32),
            pltpu.VMEM((N,), jnp.int32),
            pltpu.VMEM((N,), jnp.int32),
        ] + [pltpu.VMEM((_RADIX * _LANES,), jnp.int32) for _ in range(_G)],
    )


def _pos_body(topidx_hbm, posflat_hbm, posout_hbm, idxv, pos_row, pbuf):
    wid = lax.axis_index("s") * 2 + lax.axis_index("c")
    b = wid // (_NW // B)
    base = (wid % (_NW // B)) * _KW
    pltpu.sync_copy(topidx_hbm.at[b, pl.ds(base, _KW)], idxv)
    pltpu.sync_copy(posflat_hbm.at[b], pos_row)
    lane = lax.iota(jnp.int32, _LANES)

    def body(i, c):
        i16 = idxv[pl.ds(i * _LANES, _LANES)]
        rows = i * _LANES + lane
        for col in range(3):
            v = plsc.load_gather(pos_row, [i16 * 3 + col])
            plsc.store_scatter(pbuf, [rows * 3 + col], v)
        return c

    lax.fori_loop(0, _KW // _LANES, body, 0)
    pltpu.sync_copy(pbuf, posout_hbm.at[b, pl.ds(base * 3, _KW * 3)])


@functools.cache
def _get_pos():
    return pl.kernel(
        _pos_body,
        out_type=jax.ShapeDtypeStruct((B, K * 3), jnp.float32),
        compiler_params=pltpu.CompilerParams(needs_layout_passes=False,
                                             use_tc_tiling_on_sc=False),
        mesh=plsc.VectorSubcoreMesh(core_axis_name="c", subcore_axis_name="s",
                                    num_cores=2, num_subcores=16),
        scratch_types=[
            pltpu.VMEM((_KW,), jnp.int32),
            pltpu.VMEM((N * 3,), jnp.float32),
            pltpu.VMEM((_KW * 3,), jnp.float32),
        ],
    )


def _gather_body(topidx_hbm, y_hbm, xflat_hbm, pos16_hbm,
                 posout_hbm, xout_hbm,
                 idxv, idx2d, y_row, s_v, xbuf, pbuf, sem1, sem2):
    wid = lax.axis_index("s") * 2 + lax.axis_index("c")
    b = wid // (_NW // B)
    base = (wid % (_NW // B)) * _KW
    pltpu.sync_copy(topidx_hbm.at[b, pl.ds(base, _KW)], idxv)
    pltpu.sync_copy(y_hbm.at[b], y_row)

    def prep_body(i, c):
        i16 = idxv[pl.ds(i * _LANES, _LANES)]
        yv = plsc.load_gather(y_row, [i16])
        s_v[pl.ds(i * _LANES, _LANES)] = 1.0 / (1.0 + jnp.exp(-yv))
        idx2d[i // 8, pl.ds((i % 8) * _LANES, _LANES)] = i16 + b * N
        return c

    lax.fori_loop(0, _KW // _LANES, prep_body, 0)

    for c in range(_KW // _GC):
        cpx = pltpu.async_copy(xflat_hbm.at[idx2d.at[c]], xbuf, sem1)
        cpp = pltpu.async_copy(pos16_hbm.at[idx2d.at[c]], pbuf, sem2)
        cpx.wait()
        cpp.wait()

        def scale_body(g, carry):
            s16 = s_v[pl.ds(c * _GC + g * _LANES, _LANES)]
            for r in range(_LANES):
                sv = jnp.broadcast_to(s16[r], (_LANES,))
                for u in range(D // _LANES):
                    sl = pl.ds(u * _LANES, _LANES)
                    xbuf[g * _LANES + r, sl] = xbuf[g * _LANES + r, sl] * sv
            return carry

        lax.fori_loop(0, _GC // _LANES, scale_body, 0)
        pltpu.sync_copy(xbuf, xout_hbm.at[b, pl.ds(base + c * _GC, _GC)])
        pltpu.sync_copy(pbuf, posout_hbm.at[b, pl.ds(base + c * _GC, _GC)])


@functools.cache
def _get_gather():
    return pl.kernel(
        _gather_body,
        out_type=(
            jax.ShapeDtypeStruct((B, K, 16), jnp.float32),
            jax.ShapeDtypeStruct((B, K, D), jnp.float32),
        ),
        compiler_params=pltpu.CompilerParams(needs_layout_passes=False,
                                             use_tc_tiling_on_sc=False),
        mesh=plsc.VectorSubcoreMesh(core_axis_name="c", subcore_axis_name="s",
                                    num_cores=2, num_subcores=16),
        scratch_types=[
            pltpu.VMEM((_KW,), jnp.int32),
            pltpu.VMEM((_KW // _GC, _GC), jnp.int32),
            pltpu.VMEM((N,), jnp.float32),
            pltpu.VMEM((_KW,), jnp.float32),
            pltpu.VMEM((_GC, D), jnp.float32),
            pltpu.VMEM((_GC, 16), jnp.float32),
            pltpu.SemaphoreType.DMA,
            pltpu.SemaphoreType.DMA,
        ],
    )


def kernel(pos, x, W, b):
    y, keys = _scores(x, W, b.reshape(1, 1))
    top_idx = _get_sort()(keys)
    pos16 = jnp.pad(pos.reshape(B * N, 3), ((0, 0), (0, 13)))
    pos_sel16, x_out = _get_gather()(top_idx, y, x.reshape(B * N, D), pos16)
    return (top_idx, pos_sel16[:, :, :3], x_out)


# speedup vs baseline: 1.1618x; 1.1618x over previous
import functools

import jax
import jax.numpy as jnp
from jax import lax
from jax.experimental import pallas as pl
from jax.experimental.pallas import tpu as pltpu
from jax.experimental.pallas import tpu_sc as plsc

B = 8
N = 32768
D = 64
K = 8192

_NBLK = 32
_BN = N // _NBLK

_LANES = 16
_CHUNK = N // _LANES
_RADIX = 256
_NW = 32
_KW = K // (_NW // B)
_GC = 64


def _score_body(x_ref, w_ref, b_ref, y_ref, k_ref):
    xb = x_ref[...].reshape(B * _BN, D)
    w = w_ref[...]
    norm = jnp.sqrt(jnp.sum(w * w))
    acc = lax.dot_general(w, xb, (((1,), (1,)), ((), ())),
                          preferred_element_type=jnp.float32)
    y = (acc + b_ref[0, 0]) / norm
    y_ref[...] = y.reshape(B, _BN)
    bits = lax.bitcast_convert_type(y, jnp.int32)
    k_ref[...] = jnp.where(bits < 0, bits, ~bits & 0x7FFFFFFF).reshape(B, _BN)


def _scores(x, w, b2):
    return pl.pallas_call(
        _score_body,
        grid=(_NBLK,),
        in_specs=[
            pl.BlockSpec((B, _BN, D), lambda j: (0, j, 0)),
            pl.BlockSpec((1, D), lambda j: (0, 0)),
            pl.BlockSpec((1, 1), lambda j: (0, 0)),
        ],
        out_specs=[
            pl.BlockSpec((B, _BN), lambda j: (0, j)),
            pl.BlockSpec((B, _BN), lambda j: (0, j)),
        ],
        out_shape=[
            jax.ShapeDtypeStruct((B, N), jnp.float32),
            jax.ShapeDtypeStruct((B, N), jnp.int32),
        ],
    )(x, w, b2)


_G = 4
_NCHUNK = _LANES * _G
_CLEN = N // _NCHUNK


def _digit_pass(keys_v, hists, lane, shift, src, dst):
    ones = jnp.ones((_LANES,), jnp.int32)
    chunk_id = [lane + g * _LANES for g in range(_G)]

    def zero_body(j, c):
        for g in range(_G):
            hists[g][pl.ds(j * _LANES, _LANES)] = (
                jnp.zeros((_LANES,), jnp.int32))
        return c

    lax.fori_loop(0, _RADIX, zero_body, 0)

    def elems(i):
        if src is None:
            es = [chunk_id[g] * _CLEN + i for g in range(_G)]
        else:
            es = [plsc.load_gather(src, [chunk_id[g] * _CLEN + i])
                  for g in range(_G)]
        ks = [plsc.load_gather(keys_v, [e]) for e in es]
        addrs = []
        for k in ks:
            ku = lax.bitcast_convert_type(k, jnp.uint32)
            d = ((ku >> jnp.uint32(shift)) & jnp.uint32(0xFF))
            addrs.append(d.astype(jnp.int32) * _LANES + lane)
        return es, addrs

    def hist_body(ii, c):
        _, a0 = elems(ii * 2)
        _, a1 = elems(ii * 2 + 1)
        for g in range(_G):
            eq = (a0[g] == a1[g])
            plsc.addupdate_scatter(hists[g], [a0[g]],
                                   ones + eq.astype(jnp.int32))
            plsc.addupdate_scatter(hists[g], [a1[g]], ones,
                                   mask=jnp.logical_not(eq))
        return c

    lax.fori_loop(0, _CLEN // 2, hist_body, 0)

    def scan_body(j, carry):
        sl = pl.ds(j * _LANES, _LANES)
        hs = [hists[g][sl] for g in range(_G)]
        incs = [plsc.cumsum(h) for h in hs]
        for g in range(_G):
            hists[g][sl] = incs[g] - hs[g] + carry
            carry = carry + incs[g][_LANES - 1]
        return carry

    lax.fori_loop(0, _RADIX, scan_body, jnp.int32(0))

    def perm_body(ii, c):
        e0, a0 = elems(ii * 2)
        e1, a1 = elems(ii * 2 + 1)
        p0 = [plsc.load_gather(hists[g], [a0[g]]) for g in range(_G)]
        p1 = [plsc.load_gather(hists[g], [a1[g]]) for g in range(_G)]
        eqs = [(a0[g] == a1[g]) for g in range(_G)]
        for g in range(_G):
            plsc.store_scatter(dst, [p0[g]], e0[g])
            plsc.store_scatter(dst, [p1[g] + eqs[g].astype(jnp.int32)],
                               e1[g])
        for g in range(_G):
            plsc.addupdate_scatter(hists[g], [a0[g]],
                                   ones + eqs[g].astype(jnp.int32))
            plsc.addupdate_scatter(hists[g], [a1[g]], ones,
                                   mask=jnp.logical_not(eqs[g]))
        return c

    lax.fori_loop(0, _CLEN // 2, perm_body, 0)


def _sort_body(keys_hbm, out_hbm, s_hbm, keys_v, idx_a, idx_b, s_buf,
               h0, h1, h2, h3):
    wid = lax.axis_index("s") * 2 + lax.axis_index("c")
    hists = [h0, h1, h2, h3]

    @pl.when(wid < B)
    def _():
        pltpu.sync_copy(keys_hbm.at[wid], keys_v)
        lane = lax.iota(jnp.int32, _LANES)
        _digit_pass(keys_v, hists, lane, 0, None, idx_a)
        _digit_pass(keys_v, hists, lane, 8, idx_a, idx_b)
        _digit_pass(keys_v, hists, lane, 16, idx_b, idx_a)
        _digit_pass(keys_v, hists, lane, 24, idx_a, idx_b)
        pltpu.sync_copy(idx_b.at[pl.ds(0, K)], out_hbm.at[wid])

        def sig_body(j, c):
            i16 = idx_b[pl.ds(j * _LANES, _LANES)]
            k = plsc.load_gather(keys_v, [i16])
            bits = jnp.where(k < 0, k, ~k & 0x7FFFFFFF)
            y = lax.bitcast_convert_type(bits, jnp.float32)
            s_buf[pl.ds(j * _LANES, _LANES)] = 1.0 / (1.0 + jnp.exp(-y))
            return c

        lax.fori_loop(0, K // _LANES, sig_body, 0)
        pltpu.sync_copy(s_buf, s_hbm.at[wid])


@functools.cache
def _get_sort():
    return pl.kernel(
        _sort_body,
        out_type=(
            jax.ShapeDtypeStruct((B, K), jnp.int32),
            jax.ShapeDtypeStruct((B, K), jnp.float32),
        ),
        compiler_params=pltpu.CompilerParams(needs_layout_passes=False,
                                             use_tc_tiling_on_sc=False),
        mesh=plsc.VectorSubcoreMesh(core_axis_name="c", subcore_axis_name="s",
                                    num_cores=2, num_subcores=16),
        scratch_types=[
            pltpu.VMEM((N,), jnp.int32),
            pltpu.VMEM((N,), jnp.int32),
            pltpu.VMEM((N,), jnp.int32),
            pltpu.VMEM((K,), jnp.float32),
        ] + [pltpu.VMEM((_RADIX * _LANES,), jnp.int32) for _ in range(_G)],
    )


def _gather_body(topidx_hbm, s_hbm, xflat_hbm, posflat_hbm,
                 posout_hbm, xout_hbm,
                 idxv, idx2d, s_v, pos_row, pbuf, xbuf0, xbuf1, sem0, sem1):
    wid = lax.axis_index("s") * 2 + lax.axis_index("c")
    b = wid // (_NW // B)
    base = (wid % (_NW // B)) * _KW
    pltpu.sync_copy(topidx_hbm.at[b, pl.ds(base, _KW)], idxv)
    pltpu.sync_copy(s_hbm.at[b, pl.ds(base, _KW)], s_v)
    pltpu.sync_copy(posflat_hbm.at[b], pos_row)
    lane = lax.iota(jnp.int32, _LANES)
    nvpr = _GC // _LANES

    def prep_body(i, c):
        i16 = idxv[pl.ds(i * _LANES, _LANES)]
        idx2d[i // nvpr, pl.ds((i % nvpr) * _LANES, _LANES)] = i16 + b * N
        rows = i * _LANES + lane
        for col in range(3):
            v = plsc.load_gather(pos_row, [i16 * 3 + col])
            plsc.store_scatter(pbuf, [rows * 3 + col], v)
        return c

    lax.fori_loop(0, _KW // _LANES, prep_body, 0)
    pltpu.sync_copy(pbuf, posout_hbm.at[b, pl.ds(base * 3, _KW * 3)])

    bufs = [xbuf0, xbuf1]
    sems = [sem0, sem1]
    nchunks = _KW // _GC
    cps = [None, None]
    cps[0] = pltpu.async_copy(xflat_hbm.at[idx2d.at[0]], bufs[0], sems[0])
    for c in range(nchunks):
        if c + 1 < nchunks:
            cps[(c + 1) % 2] = pltpu.async_copy(
                xflat_hbm.at[idx2d.at[c + 1]], bufs[(c + 1) % 2],
                sems[(c + 1) % 2])
        cps[c % 2].wait()
        xbuf = bufs[c % 2]

        def scale_body(g, carry):
            s16 = s_v[pl.ds(c * _GC + g * _LANES, _LANES)]
            for r in range(_LANES):
                sv = jnp.broadcast_to(s16[r], (_LANES,))
                for u in range(D // _LANES):
                    sl = pl.ds(u * _LANES, _LANES)
                    xbuf[g * _LANES + r, sl] = xbuf[g * _LANES + r, sl] * sv
            return carry

        lax.fori_loop(0, _GC // _LANES, scale_body, 0)
        pltpu.sync_copy(xbuf, xout_hbm.at[b, pl.ds(base + c * _GC, _GC)])


@functools.cache
def _get_gather():
    return pl.kernel(
        _gather_body,
        out_type=(
            jax.ShapeDtypeStruct((B, K * 3), jnp.float32),
            jax.ShapeDtypeStruct((B, K, D), jnp.float32),
        ),
        compiler_params=pltpu.CompilerParams(needs_layout_passes=False,
                                             use_tc_tiling_on_sc=False),
        mesh=plsc.VectorSubcoreMesh(core_axis_name="c", subcore_axis_name="s",
                                    num_cores=2, num_subcores=16),
        scratch_types=[
            pltpu.VMEM((_KW,), jnp.int32),
            pltpu.VMEM((_KW // _GC, _GC), jnp.int32),
            pltpu.VMEM((_KW,), jnp.float32),
            pltpu.VMEM((N * 3,), jnp.float32),
            pltpu.VMEM((_KW * 3,), jnp.float32),
            pltpu.VMEM((_GC, D), jnp.float32),
            pltpu.VMEM((_GC, D), jnp.float32),
            pltpu.SemaphoreType.DMA,
            pltpu.SemaphoreType.DMA,
        ],
    )


def kernel(pos, x, W, b):
    y, keys = _scores(x, W, b.reshape(1, 1))
    top_idx, s = _get_sort()(keys)
    pos_flat, x_out = _get_gather()(top_idx, s, x.reshape(B * N, D),
                                    pos.reshape(B, N * 3))
    return (top_idx, pos_flat.reshape(B, K, 3), x_out)


# speedup vs baseline: 1.1795x; 1.0152x over previous
import functools

import jax
import jax.numpy as jnp
from jax import lax
from jax.experimental import pallas as pl
from jax.experimental.pallas import tpu as pltpu
from jax.experimental.pallas import tpu_sc as plsc

B = 8
N = 32768
D = 64
K = 8192

_NBLK = 16
_BN = N // _NBLK

_LANES = 16
_CHUNK = N // _LANES
_RADIX = 256
_NW = 32
_KW = K // (_NW // B)
_GC = 64


def _score_body(x_ref, w_ref, b_ref, k_ref):
    xb = x_ref[...].reshape(B * _BN, D)
    w = w_ref[...]
    norm = jnp.sqrt(jnp.sum(w * w))
    acc = lax.dot_general(w, xb, (((1,), (1,)), ((), ())),
                          preferred_element_type=jnp.float32)
    y = (acc + b_ref[0, 0]) / norm
    bits = lax.bitcast_convert_type(y, jnp.int32)
    k_ref[...] = jnp.where(bits < 0, bits, ~bits & 0x7FFFFFFF).reshape(B, _BN)


def _scores(x, w, b2):
    return pl.pallas_call(
        _score_body,
        grid=(_NBLK,),
        in_specs=[
            pl.BlockSpec((B, _BN, D), lambda j: (0, j, 0)),
            pl.BlockSpec((1, D), lambda j: (0, 0)),
            pl.BlockSpec((1, 1), lambda j: (0, 0)),
        ],
        out_specs=[
            pl.BlockSpec((B, _BN), lambda j: (0, j)),
        ],
        out_shape=[
            jax.ShapeDtypeStruct((B, N), jnp.int32),
        ],
    )(x, w, b2)


_G = 4
_NCHUNK = _LANES * _G
_CLEN = N // _NCHUNK


def _digit_pass(keys_v, hists, lane, shift, src, dst):
    ones = jnp.ones((_LANES,), jnp.int32)
    chunk_id = [lane + g * _LANES for g in range(_G)]

    def zero_body(j, c):
        for g in range(_G):
            hists[g][pl.ds(j * _LANES, _LANES)] = (
                jnp.zeros((_LANES,), jnp.int32))
        return c

    lax.fori_loop(0, _RADIX, zero_body, 0)

    def elems(i):
        if src is None:
            es = [chunk_id[g] * _CLEN + i for g in range(_G)]
        else:
            es = [plsc.load_gather(src, [chunk_id[g] * _CLEN + i])
                  for g in range(_G)]
        ks = [plsc.load_gather(keys_v, [e]) for e in es]
        addrs = []
        for k in ks:
            ku = lax.bitcast_convert_type(k, jnp.uint32)
            d = ((ku >> jnp.uint32(shift)) & jnp.uint32(0xFF))
            addrs.append(d.astype(jnp.int32) * _LANES + lane)
        return es, addrs

    def hist_body(ii, c):
        _, a0 = elems(ii * 2)
        _, a1 = elems(ii * 2 + 1)
        for g in range(_G):
            eq = (a0[g] == a1[g])
            plsc.addupdate_scatter(hists[g], [a0[g]],
                                   ones + eq.astype(jnp.int32))
            plsc.addupdate_scatter(hists[g], [a1[g]], ones,
                                   mask=jnp.logical_not(eq))
        return c

    lax.fori_loop(0, _CLEN // 2, hist_body, 0)

    def scan_body(j, carry):
        sl = pl.ds(j * _LANES, _LANES)
        hs = [hists[g][sl] for g in range(_G)]
        incs = [plsc.cumsum(h) for h in hs]
        for g in range(_G):
            hists[g][sl] = incs[g] - hs[g] + carry
            carry = carry + incs[g][_LANES - 1]
        return carry

    lax.fori_loop(0, _RADIX, scan_body, jnp.int32(0))

    def perm_body(ii, c):
        e0, a0 = elems(ii * 2)
        e1, a1 = elems(ii * 2 + 1)
        p0 = [plsc.load_gather(hists[g], [a0[g]]) for g in range(_G)]
        p1 = [plsc.load_gather(hists[g], [a1[g]]) for g in range(_G)]
        eqs = [(a0[g] == a1[g]) for g in range(_G)]
        for g in range(_G):
            plsc.store_scatter(dst, [p0[g]], e0[g])
            plsc.store_scatter(dst, [p1[g] + eqs[g].astype(jnp.int32)],
                               e1[g])
        for g in range(_G):
            plsc.addupdate_scatter(hists[g], [a0[g]],
                                   ones + eqs[g].astype(jnp.int32))
            plsc.addupdate_scatter(hists[g], [a1[g]], ones,
                                   mask=jnp.logical_not(eqs[g]))
        return c

    lax.fori_loop(0, _CLEN // 2, perm_body, 0)


def _sort_body(keys_hbm, out_hbm, s_hbm, keys_v, idx_a, idx_b, s_buf,
               h0, h1, h2, h3):
    wid = lax.axis_index("s") * 2 + lax.axis_index("c")
    hists = [h0, h1, h2, h3]

    @pl.when(wid < B)
    def _():
        pltpu.sync_copy(keys_hbm.at[wid], keys_v)
        lane = lax.iota(jnp.int32, _LANES)
        _digit_pass(keys_v, hists, lane, 0, None, idx_a)
        _digit_pass(keys_v, hists, lane, 8, idx_a, idx_b)
        _digit_pass(keys_v, hists, lane, 16, idx_b, idx_a)
        _digit_pass(keys_v, hists, lane, 24, idx_a, idx_b)
        pltpu.sync_copy(idx_b.at[pl.ds(0, K)], out_hbm.at[wid])

        def sig_body(j, c):
            i16 = idx_b[pl.ds(j * _LANES, _LANES)]
            k = plsc.load_gather(keys_v, [i16])
            bits = jnp.where(k < 0, k, ~k & 0x7FFFFFFF)
            y = lax.bitcast_convert_type(bits, jnp.float32)
            s_buf[pl.ds(j * _LANES, _LANES)] = 1.0 / (1.0 + jnp.exp(-y))
            return c

        lax.fori_loop(0, K // _LANES, sig_body, 0)
        pltpu.sync_copy(s_buf, s_hbm.at[wid])


@functools.cache
def _get_sort():
    return pl.kernel(
        _sort_body,
        out_type=(
            jax.ShapeDtypeStruct((B, K), jnp.int32),
            jax.ShapeDtypeStruct((B, K), jnp.float32),
        ),
        compiler_params=pltpu.CompilerParams(needs_layout_passes=False,
                                             use_tc_tiling_on_sc=False),
        mesh=plsc.VectorSubcoreMesh(core_axis_name="c", subcore_axis_name="s",
                                    num_cores=2, num_subcores=16),
        scratch_types=[
            pltpu.VMEM((N,), jnp.int32),
            pltpu.VMEM((N,), jnp.int32),
            pltpu.VMEM((N,), jnp.int32),
            pltpu.VMEM((K,), jnp.float32),
        ] + [pltpu.VMEM((_RADIX * _LANES,), jnp.int32) for _ in range(_G)],
    )


def _gather_body(topidx_hbm, s_hbm, xflat_hbm, posflat_hbm,
                 posout_hbm, xout_hbm,
                 idxv, idx2d, s_v, pos_row, pbuf, xbuf0, xbuf1, sem0, sem1):
    wid = lax.axis_index("s") * 2 + lax.axis_index("c")
    b = wid // (_NW // B)
    base = (wid % (_NW // B)) * _KW
    pltpu.sync_copy(topidx_hbm.at[b, pl.ds(base, _KW)], idxv)
    pltpu.sync_copy(s_hbm.at[b, pl.ds(base, _KW)], s_v)
    pltpu.sync_copy(posflat_hbm.at[b], pos_row)
    lane = lax.iota(jnp.int32, _LANES)
    nvpr = _GC // _LANES

    def prep_body(i, c):
        i16 = idxv[pl.ds(i * _LANES, _LANES)]
        idx2d[i // nvpr, pl.ds((i % nvpr) * _LANES, _LANES)] = i16 + b * N
        rows = i * _LANES + lane
        for col in range(3):
            v = plsc.load_gather(pos_row, [i16 * 3 + col])
            plsc.store_scatter(pbuf, [rows * 3 + col], v)
        return c

    lax.fori_loop(0, _KW // _LANES, prep_body, 0)
    pltpu.sync_copy(pbuf, posout_hbm.at[b, pl.ds(base * 3, _KW * 3)])

    bufs = [xbuf0, xbuf1]
    sems = [sem0, sem1]
    nchunks = _KW // _GC
    cps = [None, None]
    cps[0] = pltpu.async_copy(xflat_hbm.at[idx2d.at[0]], bufs[0], sems[0])
    for c in range(nchunks):
        if c + 1 < nchunks:
            cps[(c + 1) % 2] = pltpu.async_copy(
                xflat_hbm.at[idx2d.at[c + 1]], bufs[(c + 1) % 2],
                sems[(c + 1) % 2])
        cps[c % 2].wait()
        xbuf = bufs[c % 2]

        def scale_body(g, carry):
            s16 = s_v[pl.ds(c * _GC + g * _LANES, _LANES)]
            for r in range(_LANES):
                sv = jnp.broadcast_to(s16[r], (_LANES,))
                for u in range(D // _LANES):
                    sl = pl.ds(u * _LANES, _LANES)
                    xbuf[g * _LANES + r, sl] = xbuf[g * _LANES + r, sl] * sv
            return carry

        lax.fori_loop(0, _GC // _LANES, scale_body, 0)
        pltpu.sync_copy(xbuf, xout_hbm.at[b, pl.ds(base + c * _GC, _GC)])


@functools.cache
def _get_gather():
    return pl.kernel(
        _gather_body,
        out_type=(
            jax.ShapeDtypeStruct((B, K * 3), jnp.float32),
            jax.ShapeDtypeStruct((B, K, D), jnp.float32),
        ),
        compiler_params=pltpu.CompilerParams(needs_layout_passes=False,
                                             use_tc_tiling_on_sc=False),
        mesh=plsc.VectorSubcoreMesh(core_axis_name="c", subcore_axis_name="s",
                                    num_cores=2, num_subcores=16),
        scratch_types=[
            pltpu.VMEM((_KW,), jnp.int32),
            pltpu.VMEM((_KW // _GC, _GC), jnp.int32),
            pltpu.VMEM((_KW,), jnp.float32),
            pltpu.VMEM((N * 3,), jnp.float32),
            pltpu.VMEM((_KW * 3,), jnp.float32),
            pltpu.VMEM((_GC, D), jnp.float32),
            pltpu.VMEM((_GC, D), jnp.float32),
            pltpu.SemaphoreType.DMA,
            pltpu.SemaphoreType.DMA,
        ],
    )


def kernel(pos, x, W, b):
    [keys] = _scores(x, W, b.reshape(1, 1))
    top_idx, s = _get_sort()(keys)
    pos_flat, x_out = _get_gather()(top_idx, s, x.reshape(B * N, D),
                                    pos.reshape(B, N * 3))
    return (top_idx, pos_flat.reshape(B, K, 3), x_out)
